# baseline (speedup 1.0000x reference)
import jax
import jax.numpy as jnp
from jax.experimental import pallas as pl
from jax.experimental.pallas import tpu as pltpu
from jax.experimental.pallas import tpu_sc as plsc

MM_HIDDEN = 128
TEXT_HIDDEN = 2048
EPS = 1e-06

GATHER_WINDOW = 256
ROW_BLOCK = 256
NUM_OUT_BUFS = 8


def _sc_gather(table, ids_flat):
    n = ids_flat.shape[0]
    ids2d = ids_flat.reshape(1, n)
    mesh = plsc.VectorSubcoreMesh(core_axis_name="core", subcore_axis_name="subcore")

    @pl.kernel(
        out_type=jax.ShapeDtypeStruct((n, MM_HIDDEN), table.dtype),
        mesh=mesh,
    )
    def gather_kernel(table_hbm, ids_hbm, out_hbm):
        def body(i_vmem, o_vmem):
            pltpu.sync_copy(table_hbm.at[i_vmem.at[0]], o_vmem)

        pltpu.emit_pipeline(
            body,
            grid=(n // GATHER_WINDOW,),
            in_specs=[pl.BlockSpec((1, GATHER_WINDOW), lambda i: (0, i))],
            out_specs=[pl.BlockSpec((GATHER_WINDOW, MM_HIDDEN), lambda i: (i, 0))],
            core_axis_name=("core", "subcore"),
            dimension_semantics=(pltpu.PARALLEL,),
        )(ids_hbm, out_hbm)

    return gather_kernel(table, ids2d)


def _tc_body(x_ref, s_ref, w_ref, o_hbm, w16_ref, g16_ref, bufs, sems):
    @pl.when(pl.program_id(0) == 0)
    def _():
        w16 = w_ref[...].astype(jnp.bfloat16)
        w16_ref[...] = w16
        g = jax.lax.dot_general(
            w16, w16, (((1,), (1,)), ((), ())),
            preferred_element_type=jnp.float32,
        )
        g16_ref[...] = g.astype(jnp.bfloat16)

    x = x_ref[...]
    inv1 = jax.lax.rsqrt(jnp.mean(x * x, axis=1, keepdims=True) + EPS)
    y32 = x * inv1 * s_ref[...]
    y = y32.astype(jnp.bfloat16)
    t = jax.lax.dot_general(
        y, g16_ref[...], (((1,), (0,)), ((), ())),
        preferred_element_type=jnp.float32,
    )
    q = jnp.sum(t * y32, axis=1, keepdims=True)
    inv2 = jax.lax.rsqrt(q / TEXT_HIDDEN + EPS)
    z = jax.lax.dot_general(
        y, w16_ref[...], (((1,), (0,)), ((), ())),
        preferred_element_type=jnp.float32,
    )

    i = pl.program_id(0)
    num_steps = pl.num_programs(0)
    slot = jax.lax.rem(i, NUM_OUT_BUFS)

    def _copy(s, step):
        return pltpu.make_async_copy(
            bufs.at[s],
            o_hbm.at[pl.ds(step * ROW_BLOCK, ROW_BLOCK), :],
            sems.at[s],
        )

    @pl.when(i >= NUM_OUT_BUFS)
    def _():
        _copy(slot, i - NUM_OUT_BUFS).wait()

    bufs[slot] = z * inv2
    _copy(slot, i).start()

    @pl.when(i == num_steps - 1)
    def _():
        @pl.loop(0, NUM_OUT_BUFS)
        def _(j):
            step = num_steps - NUM_OUT_BUFS + j
            _copy(jax.lax.rem(step, NUM_OUT_BUFS), step).wait()


def _tc_norm_proj_norm(gathered, scale, weight):
    n = gathered.shape[0]
    return pl.pallas_call(
        _tc_body,
        grid=(n // ROW_BLOCK,),
        in_specs=[
            pl.BlockSpec((ROW_BLOCK, MM_HIDDEN), lambda i: (i, 0)),
            pl.BlockSpec((1, MM_HIDDEN), lambda i: (0, 0)),
            pl.BlockSpec((MM_HIDDEN, TEXT_HIDDEN), lambda i: (0, 0)),
        ],
        out_specs=pl.BlockSpec(memory_space=pl.ANY),
        out_shape=jax.ShapeDtypeStruct((n, TEXT_HIDDEN), jnp.float32),
        scratch_shapes=[
            pltpu.VMEM((MM_HIDDEN, TEXT_HIDDEN), jnp.bfloat16),
            pltpu.VMEM((MM_HIDDEN, MM_HIDDEN), jnp.bfloat16),
            pltpu.VMEM((NUM_OUT_BUFS, ROW_BLOCK, TEXT_HIDDEN), jnp.float32),
            pltpu.SemaphoreType.DMA((NUM_OUT_BUFS,)),
        ],
        compiler_params=pltpu.CompilerParams(
            dimension_semantics=("arbitrary",),
        ),
    )(gathered, scale.reshape(1, MM_HIDDEN), weight)


def kernel(input_ids, embedding_table, hard_norm_scale, projection_weight):
    b, s = input_ids.shape
    ids_flat = input_ids.reshape(b * s)
    gathered = _sc_gather(embedding_table, ids_flat)
    out = _tc_norm_proj_norm(gathered, hard_norm_scale, projection_weight)
    return out.reshape(b, s, TEXT_HIDDEN)


# speedup vs baseline: 1.2579x; 1.2579x over previous
import jax
import jax.numpy as jnp
from jax.experimental import pallas as pl
from jax.experimental.pallas import tpu as pltpu
from jax.experimental.pallas import tpu_sc as plsc

MM_HIDDEN = 128
TEXT_HIDDEN = 2048
EPS = 1e-06

GATHER_WINDOW = 256
ROW_BLOCK = 1024
NUM_OUT_BUFS = 4


def _sc_gather(table, ids_flat):
    n = ids_flat.shape[0]
    ids2d = ids_flat.reshape(1, n)
    mesh = plsc.VectorSubcoreMesh(core_axis_name="core", subcore_axis_name="subcore")

    @pl.kernel(
        out_type=jax.ShapeDtypeStruct((n, MM_HIDDEN), table.dtype),
        mesh=mesh,
    )
    def gather_kernel(table_hbm, ids_hbm, out_hbm):
        def body(i_vmem, o_vmem):
            pltpu.sync_copy(table_hbm.at[i_vmem.at[0]], o_vmem)

        pltpu.emit_pipeline(
            body,
            grid=(n // GATHER_WINDOW,),
            in_specs=[pl.BlockSpec((1, GATHER_WINDOW), lambda i: (0, i))],
            out_specs=[pl.BlockSpec((GATHER_WINDOW, MM_HIDDEN), lambda i: (i, 0))],
            core_axis_name=("core", "subcore"),
            dimension_semantics=(pltpu.PARALLEL,),
        )(ids_hbm, out_hbm)

    return gather_kernel(table, ids2d)


def _tc_body(x_ref, s_ref, w_ref, o_hbm, w16_ref, g16_ref, bufs, sems):
    @pl.when(pl.program_id(0) == 0)
    def _():
        w16 = w_ref[...].astype(jnp.bfloat16)
        w16_ref[...] = w16
        g = jax.lax.dot_general(
            w16, w16, (((1,), (1,)), ((), ())),
            preferred_element_type=jnp.float32,
        )
        g16_ref[...] = g.astype(jnp.bfloat16)

    x = x_ref[...]
    inv1 = jax.lax.rsqrt(jnp.mean(x * x, axis=1, keepdims=True) + EPS)
    y32 = x * inv1 * s_ref[...]
    y = y32.astype(jnp.bfloat16)
    t = jax.lax.dot_general(
        y, g16_ref[...], (((1,), (0,)), ((), ())),
        preferred_element_type=jnp.float32,
    )
    q = jnp.sum(t * y32, axis=1, keepdims=True)
    inv2 = jax.lax.rsqrt(q / TEXT_HIDDEN + EPS)
    z = jax.lax.dot_general(
        y, w16_ref[...], (((1,), (0,)), ((), ())),
        preferred_element_type=jnp.float32,
    )

    i = pl.program_id(0)
    num_steps = pl.num_programs(0)
    slot = jax.lax.rem(i, NUM_OUT_BUFS)

    def _copy(s, step):
        return pltpu.make_async_copy(
            bufs.at[s],
            o_hbm.at[pl.ds(step * ROW_BLOCK, ROW_BLOCK), :],
            sems.at[s],
        )

    @pl.when(i >= NUM_OUT_BUFS)
    def _():
        _copy(slot, i - NUM_OUT_BUFS).wait()

    bufs[slot] = z * inv2
    _copy(slot, i).start()

    @pl.when(i == num_steps - 1)
    def _():
        @pl.loop(0, NUM_OUT_BUFS)
        def _(j):
            step = num_steps - NUM_OUT_BUFS + j
            _copy(jax.lax.rem(step, NUM_OUT_BUFS), step).wait()


def _tc_norm_proj_norm(gathered, scale, weight):
    n = gathered.shape[0]
    return pl.pallas_call(
        _tc_body,
        grid=(n // ROW_BLOCK,),
        in_specs=[
            pl.BlockSpec((ROW_BLOCK, MM_HIDDEN), lambda i: (i, 0)),
            pl.BlockSpec((1, MM_HIDDEN), lambda i: (0, 0)),
            pl.BlockSpec((MM_HIDDEN, TEXT_HIDDEN), lambda i: (0, 0)),
        ],
        out_specs=pl.BlockSpec(memory_space=pl.ANY),
        out_shape=jax.ShapeDtypeStruct((n, TEXT_HIDDEN), jnp.float32),
        scratch_shapes=[
            pltpu.VMEM((MM_HIDDEN, TEXT_HIDDEN), jnp.bfloat16),
            pltpu.VMEM((MM_HIDDEN, MM_HIDDEN), jnp.bfloat16),
            pltpu.VMEM((NUM_OUT_BUFS, ROW_BLOCK, TEXT_HIDDEN), jnp.float32),
            pltpu.SemaphoreType.DMA((NUM_OUT_BUFS,)),
        ],
        compiler_params=pltpu.CompilerParams(
            dimension_semantics=("arbitrary",),
        ),
    )(gathered, scale.reshape(1, MM_HIDDEN), weight)


def kernel(input_ids, embedding_table, hard_norm_scale, projection_weight):
    b, s = input_ids.shape
    ids_flat = input_ids.reshape(b * s)
    gathered = _sc_gather(embedding_table, ids_flat)
    out = _tc_norm_proj_norm(gathered, hard_norm_scale, projection_weight)
    return out.reshape(b, s, TEXT_HIDDEN)


# speedup vs baseline: 1.2580x; 1.0001x over previous
import jax
import jax.numpy as jnp
from jax.experimental import pallas as pl
from jax.experimental.pallas import tpu as pltpu
from jax.experimental.pallas import tpu_sc as plsc

MM_HIDDEN = 128
TEXT_HIDDEN = 2048
EPS = 1e-06

GATHER_WINDOW = 256
ROW_BLOCK = 1024
NUM_OUT_BUFS = 4


def _sc_gather(table, ids_flat):
    n = ids_flat.shape[0]
    ids2d = ids_flat.reshape(1, n)
    mesh = plsc.VectorSubcoreMesh(core_axis_name="core", subcore_axis_name="subcore")

    @pl.kernel(
        out_type=jax.ShapeDtypeStruct((n, MM_HIDDEN), table.dtype),
        mesh=mesh,
    )
    def gather_kernel(table_hbm, ids_hbm, out_hbm):
        def body(i_vmem, o_vmem):
            pltpu.sync_copy(table_hbm.at[i_vmem.at[0]], o_vmem)

        pltpu.emit_pipeline(
            body,
            grid=(n // GATHER_WINDOW,),
            in_specs=[pl.BlockSpec((1, GATHER_WINDOW), lambda i: (0, i))],
            out_specs=[pl.BlockSpec((GATHER_WINDOW, MM_HIDDEN), lambda i: (i, 0))],
            core_axis_name=("core", "subcore"),
            dimension_semantics=(pltpu.PARALLEL,),
        )(ids_hbm, out_hbm)

    return gather_kernel(table, ids2d)


def _tc_body(x_ref, s_ref, w_ref, o_hbm, w16_ref, g16_ref, bufs, sems):
    @pl.when(pl.program_id(0) == 0)
    def _():
        w16 = w_ref[...].astype(jnp.bfloat16)
        w16_ref[...] = w16
        g = jax.lax.dot_general(
            w16, w16, (((1,), (1,)), ((), ())),
            preferred_element_type=jnp.float32,
        )
        g16_ref[...] = g.astype(jnp.bfloat16)

    x = x_ref[...]
    inv1 = jax.lax.rsqrt(jnp.mean(x * x, axis=1, keepdims=True) + EPS)
    y32 = x * inv1 * s_ref[...]
    y = y32.astype(jnp.bfloat16)
    t = jax.lax.dot_general(
        y, g16_ref[...], (((1,), (0,)), ((), ())),
        preferred_element_type=jnp.float32,
    )
    q = jnp.sum(t * y32, axis=1, keepdims=True)
    inv2 = jax.lax.rsqrt(q / TEXT_HIDDEN + EPS)
    y2 = (y32 * inv2).astype(jnp.bfloat16)
    z = jax.lax.dot_general(
        y2, w16_ref[...], (((1,), (0,)), ((), ())),
        preferred_element_type=jnp.float32,
    )

    i = pl.program_id(0)
    num_steps = pl.num_programs(0)
    slot = jax.lax.rem(i, NUM_OUT_BUFS)

    def _copy(s, step):
        return pltpu.make_async_copy(
            bufs.at[s],
            o_hbm.at[pl.ds(step * ROW_BLOCK, ROW_BLOCK), :],
            sems.at[s],
        )

    @pl.when(i >= NUM_OUT_BUFS)
    def _():
        _copy(slot, i - NUM_OUT_BUFS).wait()

    bufs[slot] = z
    _copy(slot, i).start()

    @pl.when(i == num_steps - 1)
    def _():
        @pl.loop(0, NUM_OUT_BUFS)
        def _(j):
            step = num_steps - NUM_OUT_BUFS + j
            _copy(jax.lax.rem(step, NUM_OUT_BUFS), step).wait()


def _tc_norm_proj_norm(gathered, scale, weight):
    n = gathered.shape[0]
    return pl.pallas_call(
        _tc_body,
        grid=(n // ROW_BLOCK,),
        in_specs=[
            pl.BlockSpec((ROW_BLOCK, MM_HIDDEN), lambda i: (i, 0)),
            pl.BlockSpec((1, MM_HIDDEN), lambda i: (0, 0)),
            pl.BlockSpec((MM_HIDDEN, TEXT_HIDDEN), lambda i: (0, 0)),
        ],
        out_specs=pl.BlockSpec(memory_space=pl.ANY),
        out_shape=jax.ShapeDtypeStruct((n, TEXT_HIDDEN), jnp.float32),
        scratch_shapes=[
            pltpu.VMEM((MM_HIDDEN, TEXT_HIDDEN), jnp.bfloat16),
            pltpu.VMEM((MM_HIDDEN, MM_HIDDEN), jnp.bfloat16),
            pltpu.VMEM((NUM_OUT_BUFS, ROW_BLOCK, TEXT_HIDDEN), jnp.float32),
            pltpu.SemaphoreType.DMA((NUM_OUT_BUFS,)),
        ],
        compiler_params=pltpu.CompilerParams(
            dimension_semantics=("arbitrary",),
        ),
    )(gathered, scale.reshape(1, MM_HIDDEN), weight)


def kernel(input_ids, embedding_table, hard_norm_scale, projection_weight):
    b, s = input_ids.shape
    ids_flat = input_ids.reshape(b * s)
    gathered = _sc_gather(embedding_table, ids_flat)
    out = _tc_norm_proj_norm(gathered, hard_norm_scale, projection_weight)
    return out.reshape(b, s, TEXT_HIDDEN)
